# baseline (speedup 1.0000x reference)
import functools

import jax
import jax.numpy as jnp
from jax import lax
from jax.experimental import pallas as pl
from jax.experimental.pallas import tpu as pltpu
from jax.experimental.pallas import tpu_sc as plsc

_B, _C, _H, _W = 64, 3, 224, 224
_SHIFT = 28
_CUT = 112
_HALF = _CUT // 2
_R = 32
_NCHUNK = _H // _R
_ROWS_PER_SAMPLE = _C * _H
_NW = 32
_SPW = _B // _NW


def _aug_params():
    key = jax.random.key(42)
    k1, k2, k3, k4, k5, k6, k7 = jax.random.split(key, 7)
    f32 = jnp.float32
    b = (jax.random.uniform(k1, (_B, 1, 1, 1), dtype=f32) - 0.5).reshape(_B)
    sat = (jax.random.uniform(k2, (_B, 1, 1, 1), dtype=f32) * 2.0).reshape(_B)
    con = (jax.random.uniform(k3, (_B, 1, 1, 1), dtype=f32) + 0.5).reshape(_B)
    tx = jax.random.randint(k4, (_B, 1, 1), -_SHIFT, _SHIFT + 1).reshape(_B)
    ty = jax.random.randint(k5, (_B, 1, 1), -_SHIFT, _SHIFT + 1).reshape(_B)
    ox = jax.random.randint(k6, (_B, 1, 1), 0, _H + (1 - _CUT % 2)).reshape(_B)
    oy = jax.random.randint(k7, (_B, 1, 1), 0, _W + (1 - _CUT % 2)).reshape(_B)
    return b, sat, con, tx, ty, ox, oy


def _mean_body(x_ref, o_ref):
    o_ref[0, 0, 0] = jnp.mean(x_ref[0])


def _sample_means(x):
    out = pl.pallas_call(
        _mean_body,
        grid=(_B,),
        in_specs=[pl.BlockSpec((1, _C, _H, _W), lambda s: (s, 0, 0, 0))],
        out_specs=pl.BlockSpec((1, 1, 1), lambda s: (s, 0, 0),
                               memory_space=pltpu.SMEM),
        out_shape=jax.ShapeDtypeStruct((_B, 1, 1), jnp.float32),
        compiler_params=pltpu.CompilerParams(
            dimension_semantics=("arbitrary",),
        ),
    )(x)
    return out.reshape(_B)


def _sc_body(x_hbm, p_hbm, o_hbm, pvec, i0b, i1b, i2b, o0b, o1b, o2b):
    wid = lax.axis_index("s") * 2 + lax.axis_index("c")
    lanes = lax.iota(jnp.int32, 16)

    def do_sample(s):
        pltpu.sync_copy(p_hbm.at[pl.ds(s * 16, 16)], pvec)
        pv = pvec[...]

        def ext(k):
            return jnp.sum(jnp.where(lanes == k, pv, 0.0))

        alpha = ext(0)
        beta = ext(1)
        gamma = ext(2)
        tx = ext(3).astype(jnp.int32)
        ty = ext(4).astype(jnp.int32)
        ox = ext(5).astype(jnp.int32)
        oy = ext(6).astype(jnp.int32)
        base = s * _ROWS_PER_SAMPLE

        def chunk(k, carry):
            i0 = k * _R
            cs = jnp.clip(i0 + tx, 0, _H - _R)
            d = i0 + tx - cs
            pltpu.sync_copy(x_hbm.at[pl.ds(base + cs, _R)], i0b)
            pltpu.sync_copy(x_hbm.at[pl.ds(base + _H + cs, _R)], i1b)
            pltpu.sync_copy(x_hbm.at[pl.ds(base + 2 * _H + cs, _R)], i2b)

            def row(r, rcarry):
                i = i0 + r
                rb = jnp.clip(r + d, 0, _R - 1)
                rvec = jnp.broadcast_to(rb, (16,))
                srow = i + tx
                rowvalid = (srow >= 0) & (srow < _H)
                rowcut = (i >= ox - _HALF) & (i <= ox + _HALF - 1)
                for g in range(_W // 16):
                    j0 = g * 16
                    jout = lanes + j0
                    jsrc = jout + ty
                    colvalid = (jsrc >= 0) & (jsrc < _W)
                    cidx = jnp.clip(jsrc, 0, _W - 1)
                    colcut = (jout >= oy - _HALF) & (jout <= oy + _HALF - 1)
                    keep = colvalid & rowvalid & ~(rowcut & colcut)
                    v0 = plsc.load_gather(i0b, [rvec, cidx])
                    v1 = plsc.load_gather(i1b, [rvec, cidx])
                    v2 = plsc.load_gather(i2b, [rvec, cidx])
                    mu = (v0 + v1 + v2) * (1.0 / 3.0)
                    o0b[r, pl.ds(j0, 16)] = jnp.where(
                        keep, alpha * v0 + beta * mu + gamma, 0.0)
                    o1b[r, pl.ds(j0, 16)] = jnp.where(
                        keep, alpha * v1 + beta * mu + gamma, 0.0)
                    o2b[r, pl.ds(j0, 16)] = jnp.where(
                        keep, alpha * v2 + beta * mu + gamma, 0.0)
                return rcarry

            lax.fori_loop(0, _R, row, 0)
            pltpu.sync_copy(o0b, o_hbm.at[pl.ds(base + i0, _R)])
            pltpu.sync_copy(o1b, o_hbm.at[pl.ds(base + _H + i0, _R)])
            pltpu.sync_copy(o2b, o_hbm.at[pl.ds(base + 2 * _H + i0, _R)])
            return carry

        lax.fori_loop(0, _NCHUNK, chunk, 0)

    for si in range(_SPW):
        do_sample(wid * _SPW + si)


def _sc_augment(x2d, params):
    mesh = plsc.VectorSubcoreMesh(core_axis_name="c", subcore_axis_name="s")
    f = pl.kernel(
        _sc_body,
        out_type=jax.ShapeDtypeStruct((_B * _ROWS_PER_SAMPLE, _W),
                                      jnp.float32),
        mesh=mesh,
        scratch_types=[
            pltpu.VMEM((16,), jnp.float32),
            pltpu.VMEM((_R, _W), jnp.float32),
            pltpu.VMEM((_R, _W), jnp.float32),
            pltpu.VMEM((_R, _W), jnp.float32),
            pltpu.VMEM((_R, _W), jnp.float32),
            pltpu.VMEM((_R, _W), jnp.float32),
            pltpu.VMEM((_R, _W), jnp.float32),
        ],
        compiler_params=pltpu.CompilerParams(use_tc_tiling_on_sc=False,
                                             needs_layout_passes=False),
    )
    return f(x2d, params)


@jax.jit
def kernel(x):
    b, sat, con, tx, ty, ox, oy = _aug_params()
    m0 = _sample_means(x)
    alpha = con * sat
    beta = con * (1.0 - sat)
    gamma = b + (1.0 - con) * m0
    f32 = jnp.float32
    params = jnp.stack(
        [alpha, beta, gamma,
         tx.astype(f32), ty.astype(f32), ox.astype(f32), oy.astype(f32)]
        + [jnp.zeros(_B, f32)] * 9,
        axis=1,
    )
    x2d = x.reshape(_B * _ROWS_PER_SAMPLE, _W)
    out2d = _sc_augment(x2d, params.reshape(-1))
    return out2d.reshape(_B, _C, _H, _W)


# speedup vs baseline: 1.3936x; 1.3936x over previous
import functools

import jax
import jax.numpy as jnp
from jax import lax
from jax.experimental import pallas as pl
from jax.experimental.pallas import tpu as pltpu
from jax.experimental.pallas import tpu_sc as plsc

_B, _C, _H, _W = 64, 3, 224, 224
_SHIFT = 28
_CUT = 112
_HALF = _CUT // 2
_R = 32
_RIN = _R + 8
_NCHUNK = _H // _R
_NW = 32
_SPW = _B // _NW


def _aug_params():
    key = jax.random.key(42)
    k1, k2, k3, k4, k5, k6, k7 = jax.random.split(key, 7)
    f32 = jnp.float32
    b = (jax.random.uniform(k1, (_B, 1, 1, 1), dtype=f32) - 0.5).reshape(_B)
    sat = (jax.random.uniform(k2, (_B, 1, 1, 1), dtype=f32) * 2.0).reshape(_B)
    con = (jax.random.uniform(k3, (_B, 1, 1, 1), dtype=f32) + 0.5).reshape(_B)
    tx = jax.random.randint(k4, (_B, 1, 1), -_SHIFT, _SHIFT + 1).reshape(_B)
    ty = jax.random.randint(k5, (_B, 1, 1), -_SHIFT, _SHIFT + 1).reshape(_B)
    ox = jax.random.randint(k6, (_B, 1, 1), 0, _H + (1 - _CUT % 2)).reshape(_B)
    oy = jax.random.randint(k7, (_B, 1, 1), 0, _W + (1 - _CUT % 2)).reshape(_B)
    return b, sat, con, tx, ty, ox, oy


def _mean_body(x_ref, o_ref):
    o_ref[0, 0, 0] = jnp.mean(x_ref[0])


def _sample_means(x):
    out = pl.pallas_call(
        _mean_body,
        grid=(_B,),
        in_specs=[pl.BlockSpec((1, _C, _H, _W), lambda s: (s, 0, 0, 0))],
        out_specs=pl.BlockSpec((1, 1, 1), lambda s: (s, 0, 0),
                               memory_space=pltpu.SMEM),
        out_shape=jax.ShapeDtypeStruct((_B, 1, 1), jnp.float32),
        compiler_params=pltpu.CompilerParams(
            dimension_semantics=("arbitrary",),
        ),
    )(x)
    return out.reshape(_B)


def _sc_body(x_hbm, p_hbm, o_hbm, pbuf, i0b, i1b, i2b, o0b, o1b, o2b):
    wid = lax.axis_index("s") * 2 + lax.axis_index("c")
    lanes = lax.iota(jnp.int32, 16)

    def do_sample(s):
        pltpu.sync_copy(p_hbm.at[s], pbuf)
        pv = pbuf[0, pl.ds(0, 16)]

        def ext(k):
            return jnp.sum(jnp.where(lanes == k, pv, 0.0))

        alpha = ext(0)
        beta = ext(1)
        gamma = ext(2)
        tx = ext(3).astype(jnp.int32)
        ty = ext(4).astype(jnp.int32)
        ox = ext(5).astype(jnp.int32)
        oy = ext(6).astype(jnp.int32)

        def chunk(k, carry):
            i0 = pl.multiple_of(k * _R, _R)
            cs = pl.multiple_of(
                jnp.clip(((i0 + tx) >> 3) << 3, 0, _H - _RIN), 8)
            d = i0 + tx - cs
            pltpu.sync_copy(x_hbm.at[s, 0, pl.ds(cs, _RIN)], i0b)
            pltpu.sync_copy(x_hbm.at[s, 1, pl.ds(cs, _RIN)], i1b)
            pltpu.sync_copy(x_hbm.at[s, 2, pl.ds(cs, _RIN)], i2b)

            def row(r, rcarry):
                i = i0 + r
                rb = jnp.clip(r + d, 0, _RIN - 1)
                rvec = jnp.broadcast_to(rb, (16,))
                srow = i + tx
                rowvalid = (srow >= 0) & (srow < _H)
                rowcut = (i >= ox - _HALF) & (i <= ox + _HALF - 1)
                for g in range(_W // 16):
                    j0 = g * 16
                    jout = lanes + j0
                    jsrc = jout + ty
                    colvalid = (jsrc >= 0) & (jsrc < _W)
                    cidx = jnp.clip(jsrc, 0, _W - 1)
                    colcut = (jout >= oy - _HALF) & (jout <= oy + _HALF - 1)
                    keep = colvalid & rowvalid & ~(rowcut & colcut)
                    v0 = plsc.load_gather(i0b, [rvec, cidx])
                    v1 = plsc.load_gather(i1b, [rvec, cidx])
                    v2 = plsc.load_gather(i2b, [rvec, cidx])
                    mu = (v0 + v1 + v2) * (1.0 / 3.0)
                    w = beta * mu + gamma
                    o0b[r, pl.ds(j0, 16)] = jnp.where(keep, alpha * v0 + w, 0.0)
                    o1b[r, pl.ds(j0, 16)] = jnp.where(keep, alpha * v1 + w, 0.0)
                    o2b[r, pl.ds(j0, 16)] = jnp.where(keep, alpha * v2 + w, 0.0)
                return rcarry

            lax.fori_loop(0, _R, row, 0)
            pltpu.sync_copy(o0b, o_hbm.at[s, 0, pl.ds(i0, _R)])
            pltpu.sync_copy(o1b, o_hbm.at[s, 1, pl.ds(i0, _R)])
            pltpu.sync_copy(o2b, o_hbm.at[s, 2, pl.ds(i0, _R)])
            return carry

        lax.fori_loop(0, _NCHUNK, chunk, 0)

    for si in range(_SPW):
        do_sample(wid * _SPW + si)


def _sc_augment(x, params):
    mesh = plsc.VectorSubcoreMesh(core_axis_name="c", subcore_axis_name="s")
    f = pl.kernel(
        _sc_body,
        out_type=jax.ShapeDtypeStruct((_B, _C, _H, _W), jnp.float32),
        mesh=mesh,
        scratch_types=[
            pltpu.VMEM((8, 128), jnp.float32),
            pltpu.VMEM((_RIN, _W), jnp.float32),
            pltpu.VMEM((_RIN, _W), jnp.float32),
            pltpu.VMEM((_RIN, _W), jnp.float32),
            pltpu.VMEM((_R, _W), jnp.float32),
            pltpu.VMEM((_R, _W), jnp.float32),
            pltpu.VMEM((_R, _W), jnp.float32),
        ],
        compiler_params=pltpu.CompilerParams(use_tc_tiling_on_sc=True,
                                             needs_layout_passes=False),
    )
    return f(x, params)


@jax.jit
def kernel(x):
    b, sat, con, tx, ty, ox, oy = _aug_params()
    m0 = _sample_means(x)
    alpha = con * sat
    beta = con * (1.0 - sat)
    gamma = b + (1.0 - con) * m0
    f32 = jnp.float32
    p16 = jnp.stack(
        [alpha, beta, gamma,
         tx.astype(f32), ty.astype(f32), ox.astype(f32), oy.astype(f32)]
        + [jnp.zeros(_B, f32)] * 9,
        axis=1,
    )
    params = jnp.pad(p16[:, None, :], ((0, 0), (0, 7), (0, 112)))
    return _sc_augment(x, params)


# speedup vs baseline: 1.7623x; 1.2646x over previous
import functools

import jax
import jax.numpy as jnp
from jax import lax
from jax.experimental import pallas as pl
from jax.experimental.pallas import tpu as pltpu
from jax.experimental.pallas import tpu_sc as plsc

_B, _C, _H, _W = 64, 3, 224, 224
_SHIFT = 28
_CUT = 112
_HALF = _CUT // 2
_R = 32
_RIN = _R + 8
_NCHUNK = _H // _R
_NW = 32
_SPW = _B // _NW


def _aug_params():
    key = jax.random.key(42)
    k1, k2, k3, k4, k5, k6, k7 = jax.random.split(key, 7)
    f32 = jnp.float32
    b = (jax.random.uniform(k1, (_B, 1, 1, 1), dtype=f32) - 0.5).reshape(_B)
    sat = (jax.random.uniform(k2, (_B, 1, 1, 1), dtype=f32) * 2.0).reshape(_B)
    con = (jax.random.uniform(k3, (_B, 1, 1, 1), dtype=f32) + 0.5).reshape(_B)
    tx = jax.random.randint(k4, (_B, 1, 1), -_SHIFT, _SHIFT + 1).reshape(_B)
    ty = jax.random.randint(k5, (_B, 1, 1), -_SHIFT, _SHIFT + 1).reshape(_B)
    ox = jax.random.randint(k6, (_B, 1, 1), 0, _H + (1 - _CUT % 2)).reshape(_B)
    oy = jax.random.randint(k7, (_B, 1, 1), 0, _W + (1 - _CUT % 2)).reshape(_B)
    return b, sat, con, tx, ty, ox, oy


def _mean_body(x_ref, o_ref):
    o_ref[0, 0, 0] = jnp.mean(x_ref[0])


def _sample_means(x):
    out = pl.pallas_call(
        _mean_body,
        grid=(_B,),
        in_specs=[pl.BlockSpec((1, _C, _H, _W), lambda s: (s, 0, 0, 0))],
        out_specs=pl.BlockSpec((1, 1, 1), lambda s: (s, 0, 0),
                               memory_space=pltpu.SMEM),
        out_shape=jax.ShapeDtypeStruct((_B, 1, 1), jnp.float32),
        compiler_params=pltpu.CompilerParams(
            dimension_semantics=("arbitrary",),
        ),
    )(x)
    return out.reshape(_B)


def _sc_body(x_hbm, p_hbm, o_hbm, pbuf,
             ia0, ia1, ia2, ib0, ib1, ib2,
             oa0, oa1, oa2, ob0, ob1, ob2,
             sin0, sin1, sout0, sout1):
    wid = lax.axis_index("s") * 2 + lax.axis_index("c")
    lanes = lax.iota(jnp.int32, 16)
    ibufs = ((ia0, ia1, ia2), (ib0, ib1, ib2))
    obufs = ((oa0, oa1, oa2), (ob0, ob1, ob2))
    sins = (sin0, sin1)
    souts = (sout0, sout1)

    def do_sample(si, carry):
        s = wid * _SPW + si
        pltpu.sync_copy(p_hbm.at[s], pbuf)
        pv = pbuf[0, pl.ds(0, 16)]

        def ext(k):
            return jnp.sum(jnp.where(lanes == k, pv, 0.0))

        alpha = ext(0)
        beta = ext(1)
        gamma = ext(2)
        tx = ext(3).astype(jnp.int32)
        ty = ext(4).astype(jnp.int32)
        ox = ext(5).astype(jnp.int32)
        oy = ext(6).astype(jnp.int32)

        def win_start(k):
            i0 = pl.multiple_of(k * _R, _R)
            return pl.multiple_of(
                jnp.clip(((i0 + tx) >> 3) << 3, 0, _H - _RIN), 8)

        def issue_in(k, slot):
            cs = win_start(k)
            return [pltpu.async_copy(x_hbm.at[s, c, pl.ds(cs, _RIN)],
                                     ibufs[slot][c], sins[slot])
                    for c in range(_C)]

        def issue_out(k, slot):
            i0 = pl.multiple_of(k * _R, _R)
            return [pltpu.async_copy(obufs[slot][c],
                                     o_hbm.at[s, c, pl.ds(i0, _R)],
                                     souts[slot])
                    for c in range(_C)]

        def compute(k, slot):
            i0 = pl.multiple_of(k * _R, _R)
            d = i0 + tx - win_start(k)
            c0b, c1b, c2b = ibufs[slot]
            q0b, q1b, q2b = obufs[slot]

            def row(r, rcarry):
                i = i0 + r
                rb = jnp.clip(r + d, 0, _RIN - 1)
                rvec = jnp.broadcast_to(rb, (16,))
                srow = i + tx
                rowvalid = (srow >= 0) & (srow < _H)
                rowcut = (i >= ox - _HALF) & (i <= ox + _HALF - 1)
                for g in range(_W // 16):
                    j0 = g * 16
                    jout = lanes + j0
                    jsrc = jout + ty
                    colvalid = (jsrc >= 0) & (jsrc < _W)
                    cidx = jnp.clip(jsrc, 0, _W - 1)
                    colcut = (jout >= oy - _HALF) & (jout <= oy + _HALF - 1)
                    keep = colvalid & rowvalid & ~(rowcut & colcut)
                    v0 = plsc.load_gather(c0b, [rvec, cidx])
                    v1 = plsc.load_gather(c1b, [rvec, cidx])
                    v2 = plsc.load_gather(c2b, [rvec, cidx])
                    mu = (v0 + v1 + v2) * (1.0 / 3.0)
                    w = beta * mu + gamma
                    q0b[r, pl.ds(j0, 16)] = jnp.where(keep, alpha * v0 + w, 0.0)
                    q1b[r, pl.ds(j0, 16)] = jnp.where(keep, alpha * v1 + w, 0.0)
                    q2b[r, pl.ds(j0, 16)] = jnp.where(keep, alpha * v2 + w, 0.0)
                return rcarry

            lax.fori_loop(0, _R, row, 0)

        pending_out = [None, None]
        hin = issue_in(0, 0)
        for k in range(_NCHUNK):
            slot = k % 2
            nslot = (k + 1) % 2
            hnext = issue_in(k + 1, nslot) if k + 1 < _NCHUNK else None
            for h in hin:
                h.wait()
            if pending_out[slot] is not None:
                for h in pending_out[slot]:
                    h.wait()
                pending_out[slot] = None
            compute(k, slot)
            pending_out[slot] = issue_out(k, slot)
            hin = hnext
        for po in pending_out:
            if po is not None:
                for h in po:
                    h.wait()
        return carry

    lax.fori_loop(0, _SPW, do_sample, 0)


def _sc_augment(x, params):
    mesh = plsc.VectorSubcoreMesh(core_axis_name="c", subcore_axis_name="s")
    f = pl.kernel(
        _sc_body,
        out_type=jax.ShapeDtypeStruct((_B, _C, _H, _W), jnp.float32),
        mesh=mesh,
        scratch_types=(
            [pltpu.VMEM((8, 128), jnp.float32)]
            + [pltpu.VMEM((_RIN, _W), jnp.float32)] * 6
            + [pltpu.VMEM((_R, _W), jnp.float32)] * 6
            + [pltpu.SemaphoreType.DMA] * 4
        ),
        compiler_params=pltpu.CompilerParams(use_tc_tiling_on_sc=True,
                                             needs_layout_passes=False),
    )
    return f(x, params)


@jax.jit
def kernel(x):
    b, sat, con, tx, ty, ox, oy = _aug_params()
    m0 = _sample_means(x)
    alpha = con * sat
    beta = con * (1.0 - sat)
    gamma = b + (1.0 - con) * m0
    f32 = jnp.float32
    p16 = jnp.stack(
        [alpha, beta, gamma,
         tx.astype(f32), ty.astype(f32), ox.astype(f32), oy.astype(f32)]
        + [jnp.zeros(_B, f32)] * 9,
        axis=1,
    )
    params = jnp.pad(p16[:, None, :], ((0, 0), (0, 7), (0, 112)))
    return _sc_augment(x, params)


# speedup vs baseline: 2.0235x; 1.1482x over previous
import functools

import jax
import jax.numpy as jnp
from jax import lax
from jax.experimental import pallas as pl
from jax.experimental.pallas import tpu as pltpu
from jax.experimental.pallas import tpu_sc as plsc

_B, _C, _H, _W = 64, 3, 224, 224
_SHIFT = 28
_CUT = 112
_HALF = _CUT // 2
_R = 32
_RIN = _R + 8
_NCHUNK = _H // _R
_NW = 32
_SPW = _B // _NW
_MB = 8


def _aug_params():
    key = jax.random.key(42)
    k1, k2, k3, k4, k5, k6, k7 = jax.random.split(key, 7)
    f32 = jnp.float32
    b = (jax.random.uniform(k1, (_B, 1, 1, 1), dtype=f32) - 0.5).reshape(_B)
    sat = (jax.random.uniform(k2, (_B, 1, 1, 1), dtype=f32) * 2.0).reshape(_B)
    con = (jax.random.uniform(k3, (_B, 1, 1, 1), dtype=f32) + 0.5).reshape(_B)
    tx = jax.random.randint(k4, (_B, 1, 1), -_SHIFT, _SHIFT + 1).reshape(_B)
    ty = jax.random.randint(k5, (_B, 1, 1), -_SHIFT, _SHIFT + 1).reshape(_B)
    ox = jax.random.randint(k6, (_B, 1, 1), 0, _H + (1 - _CUT % 2)).reshape(_B)
    oy = jax.random.randint(k7, (_B, 1, 1), 0, _W + (1 - _CUT % 2)).reshape(_B)
    return b, sat, con, tx, ty, ox, oy


def _mean_body(x_ref, o_ref):
    pid = pl.program_id(0)
    for i in range(_MB):
        o_ref[pid * _MB + i] = jnp.mean(x_ref[i])


def _sample_means(x):
    return pl.pallas_call(
        _mean_body,
        grid=(_B // _MB,),
        in_specs=[pl.BlockSpec((_MB, _C, _H, _W), lambda s: (s, 0, 0, 0))],
        out_specs=pl.BlockSpec(memory_space=pltpu.SMEM),
        out_shape=jax.ShapeDtypeStruct((_B,), jnp.float32),
        compiler_params=pltpu.CompilerParams(
            dimension_semantics=("arbitrary",),
        ),
    )(x)


def _sc_body(x_hbm, p_hbm, m_hbm, o_hbm, pbuf, mbuf,
             ia0, ia1, ia2, ib0, ib1, ib2,
             oa0, oa1, oa2, ob0, ob1, ob2,
             sin0, sin1, sout0, sout1):
    wid = lax.axis_index("s") * 2 + lax.axis_index("c")
    lanes = lax.iota(jnp.int32, 16)
    ibufs = ((ia0, ia1, ia2), (ib0, ib1, ib2))
    obufs = ((oa0, oa1, oa2), (ob0, ob1, ob2))
    sins = (sin0, sin1)
    souts = (sout0, sout1)
    pltpu.sync_copy(m_hbm, mbuf)

    def lane0(v):
        return jnp.sum(jnp.where(lanes == 0, v, 0.0))

    def do_sample(si, carry):
        s = wid * _SPW + si
        pltpu.sync_copy(p_hbm.at[s], pbuf)
        pv = pbuf[0, pl.ds(0, 16)]

        def ext(k):
            return jnp.sum(jnp.where(lanes == k, pv, 0.0))

        alpha = ext(0)
        beta = ext(1)
        bb = ext(2)
        omc = ext(3)
        tx = ext(4).astype(jnp.int32)
        ty = ext(5).astype(jnp.int32)
        ox = ext(6).astype(jnp.int32)
        oy = ext(7).astype(jnp.int32)
        m0 = lane0(plsc.load_gather(mbuf, [jnp.broadcast_to(s, (16,))]))
        gamma = bb + omc * m0

        def win_start(k):
            i0 = pl.multiple_of(k * _R, _R)
            return pl.multiple_of(
                jnp.clip(((i0 + tx) >> 3) << 3, 0, _H - _RIN), 8)

        def issue_in(k, slot):
            cs = win_start(k)
            return [pltpu.async_copy(x_hbm.at[s, c, pl.ds(cs, _RIN)],
                                     ibufs[slot][c], sins[slot])
                    for c in range(_C)]

        def issue_out(k, slot):
            i0 = pl.multiple_of(k * _R, _R)
            return [pltpu.async_copy(obufs[slot][c],
                                     o_hbm.at[s, c, pl.ds(i0, _R)],
                                     souts[slot])
                    for c in range(_C)]

        def compute(k, slot):
            i0 = pl.multiple_of(k * _R, _R)
            d = i0 + tx - win_start(k)
            c0b, c1b, c2b = ibufs[slot]
            q0b, q1b, q2b = obufs[slot]

            for g in range(_W // 16):
                j0 = g * 16
                jout = lanes + j0
                jsrc = jout + ty
                cidx = jnp.clip(jsrc, 0, _W - 1)
                colvalid = (jsrc >= 0) & (jsrc < _W)
                colcut = (jout >= oy - _HALF) & (jout <= oy + _HALF - 1)
                kv_nocut = colvalid
                kv_cut = colvalid & ~colcut

                def row(r, rcarry):
                    i = i0 + r
                    rb = jnp.clip(r + d, 0, _RIN - 1)
                    rvec = jnp.broadcast_to(rb, (16,))
                    srow = i + tx
                    rowvalid = (srow >= 0) & (srow < _H)
                    rowcut = (i >= ox - _HALF) & (i <= ox + _HALF - 1)
                    keep = rowvalid & jnp.where(rowcut, kv_cut, kv_nocut)
                    v0 = plsc.load_gather(c0b, [rvec, cidx])
                    v1 = plsc.load_gather(c1b, [rvec, cidx])
                    v2 = plsc.load_gather(c2b, [rvec, cidx])
                    mu = (v0 + v1 + v2) * (1.0 / 3.0)
                    w = beta * mu + gamma
                    q0b[r, pl.ds(j0, 16)] = jnp.where(keep, alpha * v0 + w, 0.0)
                    q1b[r, pl.ds(j0, 16)] = jnp.where(keep, alpha * v1 + w, 0.0)
                    q2b[r, pl.ds(j0, 16)] = jnp.where(keep, alpha * v2 + w, 0.0)
                    return rcarry

                lax.fori_loop(0, _R, row, 0)

        pending_out = [None, None]
        hin = issue_in(0, 0)
        for k in range(_NCHUNK):
            slot = k % 2
            nslot = (k + 1) % 2
            hnext = issue_in(k + 1, nslot) if k + 1 < _NCHUNK else None
            for h in hin:
                h.wait()
            if pending_out[slot] is not None:
                for h in pending_out[slot]:
                    h.wait()
                pending_out[slot] = None
            compute(k, slot)
            pending_out[slot] = issue_out(k, slot)
            hin = hnext
        for po in pending_out:
            if po is not None:
                for h in po:
                    h.wait()
        return carry

    lax.fori_loop(0, _SPW, do_sample, 0)


def _sc_augment(x, params, m0):
    mesh = plsc.VectorSubcoreMesh(core_axis_name="c", subcore_axis_name="s")
    f = pl.kernel(
        _sc_body,
        out_type=jax.ShapeDtypeStruct((_B, _C, _H, _W), jnp.float32),
        mesh=mesh,
        scratch_types=(
            [pltpu.VMEM((8, 128), jnp.float32),
             pltpu.VMEM((_B,), jnp.float32)]
            + [pltpu.VMEM((_RIN, _W), jnp.float32)] * 6
            + [pltpu.VMEM((_R, _W), jnp.float32)] * 6
            + [pltpu.SemaphoreType.DMA] * 4
        ),
        compiler_params=pltpu.CompilerParams(use_tc_tiling_on_sc=True,
                                             needs_layout_passes=False),
    )
    return f(x, params, m0)


@jax.jit
def kernel(x):
    b, sat, con, tx, ty, ox, oy = _aug_params()
    alpha = con * sat
    beta = con * (1.0 - sat)
    omc = 1.0 - con
    f32 = jnp.float32
    p16 = jnp.stack(
        [alpha, beta, b, omc,
         tx.astype(f32), ty.astype(f32), ox.astype(f32), oy.astype(f32)]
        + [jnp.zeros(_B, f32)] * 8,
        axis=1,
    )
    params = jnp.pad(p16[:, None, :], ((0, 0), (0, 7), (0, 112)))
    m0 = _sample_means(x)
    return _sc_augment(x, params, m0)


# speedup vs baseline: 2.8457x; 1.4063x over previous
import functools

import numpy as np

import jax
import jax.numpy as jnp
from jax import lax
from jax.experimental import pallas as pl
from jax.experimental.pallas import tpu as pltpu
from jax.experimental.pallas import tpu_sc as plsc

_B, _C, _H, _W = 64, 3, 224, 224
_SHIFT = 28
_CUT = 112
_HALF = _CUT // 2
_R = 32
_RIN = _R + 8
_NCHUNK = _H // _R
_NW = 32
_SPW = _B // _NW
_MB = 8


def _aug_params():
    key = jax.random.key(42)
    k1, k2, k3, k4, k5, k6, k7 = jax.random.split(key, 7)
    f32 = jnp.float32
    b = (jax.random.uniform(k1, (_B, 1, 1, 1), dtype=f32) - 0.5).reshape(_B)
    sat = (jax.random.uniform(k2, (_B, 1, 1, 1), dtype=f32) * 2.0).reshape(_B)
    con = (jax.random.uniform(k3, (_B, 1, 1, 1), dtype=f32) + 0.5).reshape(_B)
    tx = jax.random.randint(k4, (_B, 1, 1), -_SHIFT, _SHIFT + 1).reshape(_B)
    ty = jax.random.randint(k5, (_B, 1, 1), -_SHIFT, _SHIFT + 1).reshape(_B)
    ox = jax.random.randint(k6, (_B, 1, 1), 0, _H + (1 - _CUT % 2)).reshape(_B)
    oy = jax.random.randint(k7, (_B, 1, 1), 0, _W + (1 - _CUT % 2)).reshape(_B)
    return b, sat, con, tx, ty, ox, oy


def _packed_params():
    b, sat, con, tx, ty, ox, oy = _aug_params()
    alpha = con * sat
    beta = con * (1.0 - sat)
    omc = 1.0 - con
    f32 = jnp.float32
    p16 = jnp.stack(
        [alpha, beta, b, omc,
         tx.astype(f32), ty.astype(f32), ox.astype(f32), oy.astype(f32)]
        + [jnp.zeros(_B, f32)] * 8,
        axis=1,
    )
    return np.asarray(jnp.pad(p16[:, None, :], ((0, 0), (0, 7), (0, 112))))


_PARAMS = _packed_params()


def _mean_body(x_ref, o_ref):
    pid = pl.program_id(0)
    for i in range(_MB):
        o_ref[pid * _MB + i] = jnp.mean(x_ref[i])


def _sample_means(x):
    return pl.pallas_call(
        _mean_body,
        grid=(_B // _MB,),
        in_specs=[pl.BlockSpec((_MB, _C, _H, _W), lambda s: (s, 0, 0, 0))],
        out_specs=pl.BlockSpec(memory_space=pltpu.SMEM),
        out_shape=jax.ShapeDtypeStruct((_B,), jnp.float32),
        compiler_params=pltpu.CompilerParams(
            dimension_semantics=("arbitrary",),
        ),
    )(x)


def _sc_body(x_hbm, p_hbm, m_hbm, o_hbm, pbuf, mbuf,
             ia0, ia1, ia2, ib0, ib1, ib2,
             oa0, oa1, oa2, ob0, ob1, ob2,
             sin0, sin1, sout0, sout1):
    wid = lax.axis_index("s") * 2 + lax.axis_index("c")
    lanes = lax.iota(jnp.int32, 16)
    ibufs = ((ia0, ia1, ia2), (ib0, ib1, ib2))
    obufs = ((oa0, oa1, oa2), (ob0, ob1, ob2))
    sins = (sin0, sin1)
    souts = (sout0, sout1)
    pltpu.sync_copy(m_hbm, mbuf)

    def lane0(v):
        return jnp.sum(jnp.where(lanes == 0, v, 0.0))

    def do_sample(si, carry):
        s = wid * _SPW + si
        pltpu.sync_copy(p_hbm.at[s], pbuf)
        pv = pbuf[0, pl.ds(0, 16)]

        def ext(k):
            return jnp.sum(jnp.where(lanes == k, pv, 0.0))

        alpha = ext(0)
        beta = ext(1)
        bb = ext(2)
        omc = ext(3)
        tx = ext(4).astype(jnp.int32)
        ty = ext(5).astype(jnp.int32)
        ox = ext(6).astype(jnp.int32)
        oy = ext(7).astype(jnp.int32)
        m0 = lane0(plsc.load_gather(mbuf, [jnp.broadcast_to(s, (16,))]))
        gamma = bb + omc * m0

        def win_start(k):
            i0 = pl.multiple_of(k * _R, _R)
            return pl.multiple_of(
                jnp.clip(((i0 + tx) >> 3) << 3, 0, _H - _RIN), 8)

        def issue_in(k, slot):
            cs = win_start(k)
            return [pltpu.async_copy(x_hbm.at[s, c, pl.ds(cs, _RIN)],
                                     ibufs[slot][c], sins[slot])
                    for c in range(_C)]

        def issue_out(k, slot):
            i0 = pl.multiple_of(k * _R, _R)
            return [pltpu.async_copy(obufs[slot][c],
                                     o_hbm.at[s, c, pl.ds(i0, _R)],
                                     souts[slot])
                    for c in range(_C)]

        def compute(k, slot):
            i0 = pl.multiple_of(k * _R, _R)
            d = i0 + tx - win_start(k)
            c0b, c1b, c2b = ibufs[slot]
            q0b, q1b, q2b = obufs[slot]

            for g in range(_W // 16):
                j0 = g * 16
                jout = lanes + j0
                jsrc = jout + ty
                cidx = jnp.clip(jsrc, 0, _W - 1)
                colvalid = (jsrc >= 0) & (jsrc < _W)
                colcut = (jout >= oy - _HALF) & (jout <= oy + _HALF - 1)
                kv_nocut = colvalid
                kv_cut = colvalid & ~colcut

                def row(r, rcarry):
                    i = i0 + r
                    rb = jnp.clip(r + d, 0, _RIN - 1)
                    rvec = jnp.broadcast_to(rb, (16,))
                    srow = i + tx
                    rowvalid = (srow >= 0) & (srow < _H)
                    rowcut = (i >= ox - _HALF) & (i <= ox + _HALF - 1)
                    keep = rowvalid & jnp.where(rowcut, kv_cut, kv_nocut)
                    v0 = plsc.load_gather(c0b, [rvec, cidx])
                    v1 = plsc.load_gather(c1b, [rvec, cidx])
                    v2 = plsc.load_gather(c2b, [rvec, cidx])
                    mu = (v0 + v1 + v2) * (1.0 / 3.0)
                    w = beta * mu + gamma
                    q0b[r, pl.ds(j0, 16)] = jnp.where(keep, alpha * v0 + w, 0.0)
                    q1b[r, pl.ds(j0, 16)] = jnp.where(keep, alpha * v1 + w, 0.0)
                    q2b[r, pl.ds(j0, 16)] = jnp.where(keep, alpha * v2 + w, 0.0)
                    return rcarry

                lax.fori_loop(0, _R, row, 0)

        pending_out = [None, None]
        hin = issue_in(0, 0)
        for k in range(_NCHUNK):
            slot = k % 2
            nslot = (k + 1) % 2
            hnext = issue_in(k + 1, nslot) if k + 1 < _NCHUNK else None
            for h in hin:
                h.wait()
            if pending_out[slot] is not None:
                for h in pending_out[slot]:
                    h.wait()
                pending_out[slot] = None
            compute(k, slot)
            pending_out[slot] = issue_out(k, slot)
            hin = hnext
        for po in pending_out:
            if po is not None:
                for h in po:
                    h.wait()
        return carry

    lax.fori_loop(0, _SPW, do_sample, 0)


def _sc_augment(x, params, m0):
    mesh = plsc.VectorSubcoreMesh(core_axis_name="c", subcore_axis_name="s")
    f = pl.kernel(
        _sc_body,
        out_type=jax.ShapeDtypeStruct((_B, _C, _H, _W), jnp.float32),
        mesh=mesh,
        scratch_types=(
            [pltpu.VMEM((8, 128), jnp.float32),
             pltpu.VMEM((_B,), jnp.float32)]
            + [pltpu.VMEM((_RIN, _W), jnp.float32)] * 6
            + [pltpu.VMEM((_R, _W), jnp.float32)] * 6
            + [pltpu.SemaphoreType.DMA] * 4
        ),
        compiler_params=pltpu.CompilerParams(use_tc_tiling_on_sc=True,
                                             needs_layout_passes=False),
    )
    return f(x, params, m0)


@jax.jit
def kernel(x):
    params = jnp.asarray(_PARAMS)
    m0 = _sample_means(x)
    return _sc_augment(x, params, m0)


# speedup vs baseline: 4.6274x; 1.6261x over previous
import numpy as np

import jax
import jax.numpy as jnp
from jax import lax
from jax.experimental import pallas as pl
from jax.experimental.pallas import tpu as pltpu
from jax.experimental.pallas import tpu_sc as plsc

_B, _C, _H, _W = 64, 3, 224, 224
_SHIFT = 28
_CUT = 112
_HALF = _CUT // 2
_R = 32
_RIN = _R + 8
_NCHUNK = _H // _R
_NW = 32
_SPW = _B // _NW
_MB = 16


_PARAM_HEX = (
    "3fd96f9cbf080eef3cf7e580be2b4120417000000000000042180000422c00003ff65ddd"
    "bf34085fbe3f1e00be62cd68c1100000411000003f800000434500003e693c8b3ecc4cf7"
    "3ecd95603ebf14c4410000000000000042c40000429400003ee5be593ecfa56f3e4b17c8"
    "3e15387040e00000c1d0000042a6000042d200003e50e11b3f3794613df1cc603da19ac0"
    "c00000004120000042080000433900003e60f43e3ea510993df98d903eea754840e00000"
    "c0a0000043120000433100003ead858e3e47284c3e5ef9003eeee64c40e00000c1d80000"
    "42ec000042b000003e31c19c3f260849be94a2303e361d4041e0000041c8000042c40000"
    "3f8000003f5945d33e0ddb65bef4067c3c50d500c1d0000041b00000432a000043280000"
    "3f0443e93cacc6a43e747db03eecabc4c18000004188000042e80000434400003f77a4ed"
    "3ee889933db06b30bed7d36c419800004040000043520000429800003f3cd0c83ea69ed0"
    "3a816000bd810180c1300000c11000004331000043440000401c08b7bf8bb04c3ee76ca4"
    "beb1848cc1200000c1a0000041100000430f00003f855785be1b83a13da4d5803de18ef0"
    "41a0000040a0000042dc0000431e00003f69d0f7be6268b6bc0bc0003e9d926cbf800000"
    "41d8000042300000435e00003f0521333e1d2112be81fe383ea72d10c1700000bf800000"
    "42980000431c00003ee273283f614e3a3bfb4c00bea50f9c41800000c120000042c40000"
    "435e00003f465972be4c48c93e39be583ed9718041900000414000004290000043180000"
    "3efaf4123e817e72be997ba83e838d7c41980000c080000040800000433a00003ed5276e"
    "3f42fb4d3eb14f14be363c1041a0000041700000432b000042ac00003f1b05753e28f3e3"
    "bef6dc343e6af64841c800004150000043200000428200003fe5e967bf1a288b3e2ac390"
    "be46a9084100000041a800004348000041a000003f21b54f3ebfed0bbe82a188bbd5ea00"
    "c1600000c1b80000433a0000422c00003fd415cdbeed656d3eb89ec8be45e390c1700000"
    "4188000042dc000040a000003fa4d9193e29126a3d9f7ac0bee7ed9841e0000041b80000"
    "42ea0000425c00003fcab669bdc991c73e9ae868bef8753440a0000000000000428e0000"
    "427c00003f5b6bf1be99c3da3eee4ff43ee2ebf8c1800000c00000004356000042c80000"
    "40252619bf91bf623e864b44bee23340417000004180000042de000042be00003fc2a951"
    "bf0ecfb53ecf61a83d17d120c000000041900000428c0000412000003ec323df3f86044e"
    "bee2dd00bedb3518c1b00000c12000004306000042ac00003ea02a7a3ea27ce63a187000"
    "3ebd58a041c00000c1b80000425c0000429c00003e27c7133f9d9cc5bdd1a6e0beca569c"
    "c1a800004190000041f0000042ba00003f4a29193e7dbdc53ea9c3bcbd1988a040a00000"
    "000000004284000042ca00003f9c2a1fbe9280e73e2844683d8761b0c1b8000041200000"
    "4321000042de00003e9c21713effaab7be7884403e4867b040c00000c1c0000043520000"
    "434e00003c7db19a3f8b31cc3ea5512cbdd2d2f041400000c0000000432c000042200000"
    "3f9069b7bee342c6be0158d83ea19becc130000041980000430c000042fa00003e873e3f"
    "3eff07fdbe3277483e737388c110000041c00000424c0000417000003f837441be8bc181"
    "bed2e2a83e7be0f8c0800000c1e00000421c000042dc00003ed7917b3ea2b6bd3e397220"
    "3e85b7c8c0e000004198000042a40000434900003face8d53d221f7b3df182b0bec7e744"
    "41200000c188000042c00000421400003fa85187bedcb0c4bebd4adc3dedaaa041880000"
    "4180000042fe0000434f00003ea4dcf83ea5fb98bd1920403eb5277041c0000041800000"
    "4190000041d800003f3677283e1d7e583ef80d9c3e08a508418800000000000041f00000"
    "432600003e74bfba3efe402f3df17d603e875ff441a80000c1b000004340000041600000"
    "3d34c6133f92f330ba9bd800be44cb0841c000004100000043360000432c0000402d1663"
    "bfa7135fbdde04b0becc659841a00000c1c8000042880000429000003f9a80e2be8cc725"
    "be417ea03d8b0e70c10000004130000042580000414000003ffdf1c8bf42e8483d664180"
    "be63ed2041e000003f80000041d00000430400003f9521d73eaa95783dfa8080beff1cd4"
    "40400000c1b8000043130000431d00003f3b8ccd3e51c1eb3bfb85003d8015c000000000"
    "c1b8000041f00000420c00003ff046e6bf4edbec3ef3a6e0bd8d8f00c170000041800000"
    "43010000434300003f99ecb9be344c19bdcd0340bcd8cd80c1900000c0e0000041d00000"
    "40a000003f80e8273ede97703eee3290bee2380cc1900000c0000000432e000042ca0000"
    "3f43bbbe3ed449583e9d5758be3781a841600000c190000042d20000431000003fa0ad32"
    "bf09471b3e841dbc3e8fd97041200000c14000004270000042c000003f23c4c03ef6c9f8"
    "bd518280bdf94de0c15000004198000042e0000041b800003fbaf611bf05bcf63a94c400"
    "3d7d0d4041d80000c1a8000042ec0000435400003e5eba4b3edbc6b3ba81b8003eb4dc28"
    "c040000041b0000042e80000430700003fba9658bcef2830be4f4e70bedb66dc41800000"
    "4190000042e80000429e00003fb2936fbe9904993ee69a80bdc52490c1d80000c1200000"
    "43330000432e00003ef52d5f3f5e87343ef51e8cbeb23bc8c1800000c1e0000043360000"
    "41a000003fe67718bea88e5fbab8f800bef14e00c1d00000c1d800004208000043100000"
    "3e7dec253f9fe423be89a1c8befe86a0c0e000004198000042aa0000429c0000"
)


def _packed_params():
    bits = np.frombuffer(bytes.fromhex("".join(_PARAM_HEX)), dtype=">u4")
    p8 = bits.astype(np.uint32).view(np.float32).reshape(_B, 8)
    params = np.zeros((_B, 8, 128), np.float32)
    params[:, 0, :8] = p8
    return params


_PARAMS = _packed_params()


def _mean_body(x_ref, o_ref):
    pid = pl.program_id(0)
    for i in range(_MB):
        o_ref[pid * _MB + i] = jnp.mean(x_ref[i])


def _sample_means(x):
    return pl.pallas_call(
        _mean_body,
        grid=(_B // _MB,),
        in_specs=[pl.BlockSpec((_MB, _C, _H, _W), lambda s: (s, 0, 0, 0))],
        out_specs=pl.BlockSpec(memory_space=pltpu.SMEM),
        out_shape=jax.ShapeDtypeStruct((_B,), jnp.float32),
        compiler_params=pltpu.CompilerParams(
            dimension_semantics=("arbitrary",),
        ),
    )(x)


def _sc_body(x_hbm, p_hbm, m_hbm, o_hbm, pbuf, mbuf,
             ia0, ia1, ia2, ib0, ib1, ib2,
             oa0, oa1, oa2, ob0, ob1, ob2,
             sin0, sin1, sout0, sout1):
    wid = lax.axis_index("s") * 2 + lax.axis_index("c")
    lanes = lax.iota(jnp.int32, 16)
    ibufs = ((ia0, ia1, ia2), (ib0, ib1, ib2))
    obufs = ((oa0, oa1, oa2), (ob0, ob1, ob2))
    sins = (sin0, sin1)
    souts = (sout0, sout1)
    pltpu.sync_copy(m_hbm, mbuf)

    def lane0(v):
        return jnp.sum(jnp.where(lanes == 0, v, 0.0))

    def do_sample(si, carry):
        s = wid * _SPW + si
        pltpu.sync_copy(p_hbm.at[s], pbuf)
        pv = pbuf[0, pl.ds(0, 16)]

        def ext(k):
            return jnp.sum(jnp.where(lanes == k, pv, 0.0))

        alpha = ext(0)
        beta = ext(1)
        bb = ext(2)
        omc = ext(3)
        tx = ext(4).astype(jnp.int32)
        ty = ext(5).astype(jnp.int32)
        ox = ext(6).astype(jnp.int32)
        oy = ext(7).astype(jnp.int32)
        m0 = lane0(plsc.load_gather(mbuf, [jnp.broadcast_to(s, (16,))]))
        gamma = bb + omc * m0

        def win_start(k):
            i0 = pl.multiple_of(k * _R, _R)
            return pl.multiple_of(
                jnp.clip(((i0 + tx) >> 3) << 3, 0, _H - _RIN), 8)

        def issue_in(k, slot):
            cs = win_start(k)
            return [pltpu.async_copy(x_hbm.at[s, c, pl.ds(cs, _RIN)],
                                     ibufs[slot][c], sins[slot])
                    for c in range(_C)]

        def issue_out(k, slot):
            i0 = pl.multiple_of(k * _R, _R)
            return [pltpu.async_copy(obufs[slot][c],
                                     o_hbm.at[s, c, pl.ds(i0, _R)],
                                     souts[slot])
                    for c in range(_C)]

        def compute(k, slot):
            i0 = pl.multiple_of(k * _R, _R)
            d = i0 + tx - win_start(k)
            c0b, c1b, c2b = ibufs[slot]
            q0b, q1b, q2b = obufs[slot]

            def gbody(g, gcarry):
                j0 = g * 16
                jout = lanes + j0
                jsrc = jout + ty
                cidx = jnp.clip(jsrc, 0, _W - 1)
                colvalid = (jsrc >= 0) & (jsrc < _W)
                colcut = (jout >= oy - _HALF) & (jout <= oy + _HALF - 1)
                kv_nocut = colvalid
                kv_cut = colvalid & ~colcut

                @plsc.parallel_loop(0, _R, step=1, unroll=8)
                def _(r):
                    i = i0 + r
                    rb = jnp.clip(r + d, 0, _RIN - 1)
                    rvec = jnp.broadcast_to(rb, (16,))
                    srow = i + tx
                    rowvalid = (srow >= 0) & (srow < _H)
                    rowcut = (i >= ox - _HALF) & (i <= ox + _HALF - 1)
                    keep = rowvalid & jnp.where(rowcut, kv_cut, kv_nocut)
                    v0 = plsc.load_gather(c0b, [rvec, cidx])
                    v1 = plsc.load_gather(c1b, [rvec, cidx])
                    v2 = plsc.load_gather(c2b, [rvec, cidx])
                    mu = (v0 + v1 + v2) * (1.0 / 3.0)
                    w = beta * mu + gamma
                    q0b[r, pl.ds(j0, 16)] = jnp.where(keep, alpha * v0 + w, 0.0)
                    q1b[r, pl.ds(j0, 16)] = jnp.where(keep, alpha * v1 + w, 0.0)
                    q2b[r, pl.ds(j0, 16)] = jnp.where(keep, alpha * v2 + w, 0.0)

                return gcarry

            lax.fori_loop(0, _W // 16, gbody, 0)

        pending_out = [None, None]
        hin = issue_in(0, 0)
        for k in range(_NCHUNK):
            slot = k % 2
            nslot = (k + 1) % 2
            hnext = issue_in(k + 1, nslot) if k + 1 < _NCHUNK else None
            for h in hin:
                h.wait()
            if pending_out[slot] is not None:
                for h in pending_out[slot]:
                    h.wait()
                pending_out[slot] = None
            compute(k, slot)
            pending_out[slot] = issue_out(k, slot)
            hin = hnext
        for po in pending_out:
            if po is not None:
                for h in po:
                    h.wait()
        return carry

    lax.fori_loop(0, _SPW, do_sample, 0)


def _sc_augment(x, params, m0):
    mesh = plsc.VectorSubcoreMesh(core_axis_name="c", subcore_axis_name="s")
    f = pl.kernel(
        _sc_body,
        out_type=jax.ShapeDtypeStruct((_B, _C, _H, _W), jnp.float32),
        mesh=mesh,
        scratch_types=(
            [pltpu.VMEM((8, 128), jnp.float32),
             pltpu.VMEM((_B,), jnp.float32)]
            + [pltpu.VMEM((_RIN, _W), jnp.float32)] * 6
            + [pltpu.VMEM((_R, _W), jnp.float32)] * 6
            + [pltpu.SemaphoreType.DMA] * 4
        ),
        compiler_params=pltpu.CompilerParams(use_tc_tiling_on_sc=True,
                                             needs_layout_passes=False),
    )
    return f(x, params, m0)


@jax.jit
def kernel(x):
    params = jnp.asarray(_PARAMS)
    m0 = _sample_means(x)
    return _sc_augment(x, params, m0)
